# baseline (speedup 1.0000x reference)
import functools

import jax
import jax.numpy as jnp
from jax import lax
from jax.experimental import pallas as pl
from jax.experimental.pallas import tpu as pltpu
from jax.experimental.pallas import tpu_sc as plsc

B, F, D = 16384, 26, 32
H = D // 2
FP = 32

_info = plsc.get_sparse_core_info()
_NC, _NS = _info.num_cores, _info.num_subcores
NW = _NC * _NS
C = B // NW
S = 128
SH = S // 2
NSUB = C // S
FS = SH * F
NB = B // S
assert C % S == 0 and B % NW == 0 and S % 32 == 0


def _tile_flat(x):
  xp = jnp.pad(x, ((0, 0), (0, FP - F)))
  return xp.reshape(NB, S, FP // 8, 8).transpose(2, 0, 3, 1).reshape(-1)


def _sc_forward(user_ids, item_ids, ufi, ufv, ifi, ifv, U, I, UF, IF):
  mesh = plsc.VectorSubcoreMesh(core_axis_name="c", subcore_axis_name="s")

  @functools.partial(
      pl.kernel,
      mesh=mesh,
      compiler_params=pltpu.CompilerParams(use_tc_tiling_on_sc=False),
      out_type=jax.ShapeDtypeStruct((B,), jnp.float32),
      scratch_types=[
          pltpu.VMEM((S,), jnp.int32),
          pltpu.VMEM((S,), jnp.int32),
          pltpu.VMEM((F * S,), jnp.int32),
          pltpu.VMEM((F * S,), jnp.float32),
          pltpu.VMEM((F * S,), jnp.int32),
          pltpu.VMEM((F * S,), jnp.float32),
          pltpu.VMEM((SH, D), jnp.float32),
          pltpu.VMEM((SH, D), jnp.float32),
          pltpu.VMEM((FS, D), jnp.float32),
          pltpu.VMEM((FS, D), jnp.float32),
          pltpu.VMEM((S,), jnp.float32),
          pltpu.SemaphoreType.DMA,
      ],
  )
  def k(uid_h, iid_h, ufi_h, ufv_h, ifi_h, ifv_h, U_h, I_h, UF_h, IF_h,
        out_h, uids_v, iids_v, ufi_v, ufv_v, ifi_v, ifv_v,
        urows_v, irows_v, ufrows_v, ifrows_v, out_v, sem):
    wid = lax.axis_index("s") * _NC + lax.axis_index("c")
    lane_iota = lax.iota(jnp.int32, 16)

    def sub(j, carry):
      base = wid * C + j * S
      blk = wid * NSUB + j
      pltpu.sync_copy(uid_h.at[pl.ds(base, S)], uids_v)
      pltpu.sync_copy(iid_h.at[pl.ds(base, S)], iids_v)
      for a in range(FP // 8):
        n = 8 * S if a < 3 else (F - 24) * S
        off = (a * NB + blk) * 8 * S
        rows = pl.ds(a * 8 * S, n)
        pltpu.sync_copy(ufi_h.at[pl.ds(off, n)], ufi_v.at[rows])
        pltpu.sync_copy(ufv_h.at[pl.ds(off, n)], ufv_v.at[rows])
        pltpu.sync_copy(ifi_h.at[pl.ds(off, n)], ifi_v.at[rows])
        pltpu.sync_copy(ifv_h.at[pl.ds(off, n)], ifv_v.at[rows])

      for h in range(2):
        hb = h * SH
        cps = [pltpu.async_copy(U_h.at[uids_v.at[pl.ds(hb, SH)]], urows_v, sem),
               pltpu.async_copy(I_h.at[iids_v.at[pl.ds(hb, SH)]], irows_v, sem)]
        for f in range(F):
          src = pl.ds(f * S + hb, SH)
          dst = pl.ds(f * SH, SH)
          cps.append(pltpu.async_copy(UF_h.at[ufi_v.at[src]], ufrows_v.at[dst], sem))
          cps.append(pltpu.async_copy(IF_h.at[ifi_v.at[src]], ifrows_v.at[dst], sem))
        for cp in cps:
          cp.wait()

        def group(bg, carry2):
          b0 = hb + bg * 16

          def lane(l, acc):
            b = bg * 16 + l
            lvec = jnp.zeros((16,), jnp.int32) + l
            u0 = urows_v[b, 0:H]
            u1 = urows_v[b, H:D]
            i0 = irows_v[b, 0:H]
            i1 = irows_v[b, H:D]
            for f in range(F):
              p = f * SH + b
              wu = ufv_v[pl.ds(f * S + b0, 16)][lvec]
              u0 = u0 + ufrows_v[p, 0:H] * wu
              u1 = u1 + ufrows_v[p, H:D] * wu
              wi = ifv_v[pl.ds(f * S + b0, 16)][lvec]
              i0 = i0 + ifrows_v[p, 0:H] * wi
              i1 = i1 + ifrows_v[p, H:D] * wi
            prod = u0 * i0 + u1 * i1
            for sh in (8, 4, 2, 1):
              prod = prod + prod[lane_iota ^ sh]
            return jnp.where(lane_iota == l, prod, acc)

          acc = lax.fori_loop(0, 16, lane, jnp.zeros((16,), jnp.float32))
          out_v[pl.ds(hb + bg * 16, 16)] = acc
          return carry2

        lax.fori_loop(0, SH // 16, group, 0)

      pltpu.sync_copy(out_v, out_h.at[pl.ds(base, S)])
      return carry

    lax.fori_loop(0, NSUB, sub, 0)

  return k(user_ids, item_ids, ufi, ufv, ifi, ifv, U, I, UF, IF)


def kernel(user_ids, item_ids, user_feature_indices, user_feature_values,
           item_feature_indices, item_feature_values, U, I, UF, IF):
  return _sc_forward(
      user_ids.astype(jnp.int32),
      item_ids.astype(jnp.int32),
      _tile_flat(user_feature_indices.astype(jnp.int32)),
      _tile_flat(user_feature_values),
      _tile_flat(item_feature_indices.astype(jnp.int32)),
      _tile_flat(item_feature_values),
      U, I, UF, IF)


# speedup vs baseline: 1.1373x; 1.1373x over previous
import functools

import jax
import jax.numpy as jnp
from jax import lax
from jax.experimental import pallas as pl
from jax.experimental.pallas import tpu as pltpu
from jax.experimental.pallas import tpu_sc as plsc

B, F, D = 16384, 26, 32
H = D // 2
FP = 32
NU = 1000000

_info = plsc.get_sparse_core_info()
_NC, _NS = _info.num_cores, _info.num_subcores
NW = _NC * _NS
C = B // NW
S = 128
SH = S // 2
NSUB = C // S
FS = SH * F
NB = B // S

KC = 32
GC = (NU + 128 * KC - 1) // (128 * KC)
TILES = GC * KC
FLAT = 4 * TILES * 1024
assert C % S == 0 and B % NW == 0 and S % 32 == 0


def _unpack_bf16(row):
  x = plsc.bitcast(row, jnp.int32)
  a = plsc.bitcast(x << 16, jnp.float32)
  b = plsc.bitcast((x >> 16) << 16, jnp.float32)
  return a, b


def _tile_flat(x):
  xp = jnp.pad(x, ((0, 0), (0, FP - F)))
  return xp.reshape(NB, S, FP // 8, 8).transpose(2, 0, 3, 1).reshape(-1)


def _tc_launder(U, I):
  def body(ut_ref, it_ref, uo_ref, io_ref):
    for c in range(KC):
      src = pl.ds(c * 128, 128)
      dst = pl.ds(c * 1024, 1024)
      uo_ref[dst] = ut_ref[:, src].reshape(1024)
      io_ref[dst] = it_ref[:, src].reshape(1024)

  return pl.pallas_call(
      body,
      grid=(4, GC),
      in_specs=[pl.BlockSpec((8, KC * 128), lambda a, c: (a, c)),
                pl.BlockSpec((8, KC * 128), lambda a, c: (a, c))],
      out_specs=[pl.BlockSpec((KC * 1024,), lambda a, c: (a * GC + c,)),
                 pl.BlockSpec((KC * 1024,), lambda a, c: (a * GC + c,))],
      out_shape=[jax.ShapeDtypeStruct((FLAT,), jnp.float32),
                 jax.ShapeDtypeStruct((FLAT,), jnp.float32)],
  )(U.T, I.T)


def _sc_forward(user_ids, item_ids, ufi, ufv, ifi, ifv, Uf, If, UF, IF):
  mesh = plsc.VectorSubcoreMesh(core_axis_name="c", subcore_axis_name="s")

  @functools.partial(
      pl.kernel,
      mesh=mesh,
      compiler_params=pltpu.CompilerParams(use_tc_tiling_on_sc=False,
                                           needs_layout_passes=False),
      out_type=jax.ShapeDtypeStruct((B,), jnp.float32),
      scratch_types=[
          pltpu.VMEM((S,), jnp.int32),
          pltpu.VMEM((S,), jnp.int32),
          pltpu.VMEM((F * S,), jnp.int32),
          pltpu.VMEM((F * S,), jnp.float32),
          pltpu.VMEM((F * S,), jnp.int32),
          pltpu.VMEM((F * S,), jnp.float32),
          pltpu.VMEM((S,), jnp.int32),
          pltpu.VMEM((S,), jnp.int32),
          pltpu.VMEM((2 * D * S,), jnp.int32),
          pltpu.VMEM((D * S,), jnp.float32),
          pltpu.VMEM((D * S,), jnp.float32),
          pltpu.VMEM((FS, D), jnp.bfloat16),
          pltpu.VMEM((FS, D), jnp.bfloat16),
          pltpu.VMEM((S,), jnp.float32),
          pltpu.SemaphoreType.DMA,
          pltpu.SemaphoreType.DMA,
      ],
  )
  def k(uid_h, iid_h, ufi_h, ufv_h, ifi_h, ifv_h, Uf_h, If_h, UF_h, IF_h,
        out_h, uids_v, iids_v, ufi_v, ufv_v, ifi_v, ifv_v,
        ub_v, ib_v, gidx_v, urowsT_v, irowsT_v, ufrows_v, ifrows_v,
        out_v, sem, sem2):
    wid = lax.axis_index("s") * _NC + lax.axis_index("c")
    lane_iota = lax.iota(jnp.int32, 16)
    wlo_idx = lane_iota * S
    whi_idx = jnp.minimum(lane_iota + 16, F - 1) * S
    col_idx = lane_iota * S

    def sub(j, carry):
      base = wid * C + j * S
      blk = wid * NSUB + j
      pltpu.sync_copy(uid_h.at[pl.ds(base, S)], uids_v)
      pltpu.sync_copy(iid_h.at[pl.ds(base, S)], iids_v)
      for a in range(FP // 8):
        n = 8 * S if a < 3 else (F - 24) * S
        off = (a * NB + blk) * 8 * S
        rows = pl.ds(a * 8 * S, n)
        pltpu.sync_copy(ufi_h.at[pl.ds(off, n)], ufi_v.at[rows])
        pltpu.sync_copy(ufv_h.at[pl.ds(off, n)], ufv_v.at[rows])
        pltpu.sync_copy(ifi_h.at[pl.ds(off, n)], ifi_v.at[rows])
        pltpu.sync_copy(ifv_h.at[pl.ds(off, n)], ifv_v.at[rows])

      for c in range(S // 16):
        sl = pl.ds(c * 16, 16)
        uid = uids_v[sl]
        ub_v[sl] = ((uid >> 7) << 10) + (uid & 127)
        iid = iids_v[sl]
        ib_v[sl] = ((iid >> 7) << 10) + (iid & 127)

      cps_ui = []
      for tbl, bvv, src_h, dst in ((0, ub_v, Uf_h, urowsT_v),
                                   (1, ib_v, If_h, irowsT_v)):
        for d in range(D):
          a, s = d // 8, d % 8
          const = a * TILES * 1024 + s * 128
          gbase = tbl * D * S + d * S
          for c in range(S // 16):
            gidx_v[pl.ds(gbase + c * 16, 16)] = bvv[pl.ds(c * 16, 16)] + const
          cps_ui.append(pltpu.async_copy(
              src_h.at[gidx_v.at[pl.ds(gbase, S)]],
              dst.at[pl.ds(d * S, S)], sem2))

      for h in range(2):
        hb = h * SH
        cps = []
        for f in range(F):
          src = pl.ds(f * S + hb, SH)
          dstf = pl.ds(f * SH, SH)
          cps.append(pltpu.async_copy(UF_h.at[ufi_v.at[src]], ufrows_v.at[dstf], sem))
          cps.append(pltpu.async_copy(IF_h.at[ifi_v.at[src]], ifrows_v.at[dstf], sem))
        if h == 0:
          for cp in cps_ui:
            cp.wait()
        for cp in cps:
          cp.wait()

        def group(bg, carry2):
          def lane(l, acc):
            b = bg * 16 + l
            bb = hb + b
            bvec = jnp.zeros((16,), jnp.int32) + bb
            u0 = plsc.load_gather(urowsT_v, [col_idx + bvec])
            u1 = plsc.load_gather(urowsT_v, [col_idx + (16 * S + bb)])
            i0 = plsc.load_gather(irowsT_v, [col_idx + bvec])
            i1 = plsc.load_gather(irowsT_v, [col_idx + (16 * S + bb)])
            uw0 = plsc.load_gather(ufv_v, [wlo_idx + bvec])
            uw1 = plsc.load_gather(ufv_v, [whi_idx + bvec])
            iw0 = plsc.load_gather(ifv_v, [wlo_idx + bvec])
            iw1 = plsc.load_gather(ifv_v, [whi_idx + bvec])
            for f in range(F):
              p = f * SH + b
              wu = uw0[f] if f < 16 else uw1[f - 16]
              ua, ub = _unpack_bf16(ufrows_v[p, :])
              u0 = u0 + ua * wu
              u1 = u1 + ub * wu
              wi = iw0[f] if f < 16 else iw1[f - 16]
              ia, ib = _unpack_bf16(ifrows_v[p, :])
              i0 = i0 + ia * wi
              i1 = i1 + ib * wi
            prod = u0 * i0 + u1 * i1
            for sh in (8, 4, 2, 1):
              prod = prod + prod[lane_iota ^ sh]
            return jnp.where(lane_iota == l, prod, acc)

          acc = lax.fori_loop(0, 16, lane, jnp.zeros((16,), jnp.float32))
          out_v[pl.ds(hb + bg * 16, 16)] = acc
          return carry2

        lax.fori_loop(0, SH // 16, group, 0)

      pltpu.sync_copy(out_v, out_h.at[pl.ds(base, S)])
      return carry

    lax.fori_loop(0, NSUB, sub, 0)

  return k(user_ids, item_ids, ufi, ufv, ifi, ifv, Uf, If, UF, IF)


def kernel(user_ids, item_ids, user_feature_indices, user_feature_values,
           item_feature_indices, item_feature_values, U, I, UF, IF):
  Uf, If = _tc_launder(U, I)
  return _sc_forward(
      user_ids.astype(jnp.int32),
      item_ids.astype(jnp.int32),
      _tile_flat(user_feature_indices.astype(jnp.int32)),
      _tile_flat(user_feature_values),
      _tile_flat(item_feature_indices.astype(jnp.int32)),
      _tile_flat(item_feature_values),
      Uf, If,
      UF.astype(jnp.bfloat16), IF.astype(jnp.bfloat16))
